# baseline (speedup 1.0000x reference)
import functools

import jax
import jax.numpy as jnp
from jax import lax
from jax.experimental import pallas as pl
from jax.experimental.pallas import tpu as pltpu

BR = 256
BC = 1024
LOG2E = 1.4426950408889634


def _leaky(x):
    return jnp.maximum(x, 0.01 * x)


def _attn_kernel(n2e_ref, e2e_ref, nf_ref, ef_ref, nfb_ref, efb_ref,
                 wn_ref, we_ref, pvn_ref, pve_ref, out_n_ref, out_e_ref,
                 part_n, part_e, partb_n, partb_e, mp_n, mp_e,
                 *, nj, ee):
    i = pl.program_id(0)
    cd = (((1,), (1,)), ((), ()))
    c0 = (((0,), (1,)), ((), ()))
    c00 = (((0,), (0,)), ((), ()))
    f32 = jnp.float32
    bf16 = jnp.bfloat16

    @pl.when(i == 0)
    def _precompute():
        wn_vec = lax.dot_general(wn_ref[...], pvn_ref[...][:, ee:], cd,
                                 preferred_element_type=f32)
        we_p = lax.dot_general(we_ref[...], pve_ref[...][:, ee:], cd,
                               preferred_element_type=f32)
        pn = jnp.dot(nf_ref[...], wn_vec, preferred_element_type=f32)
        pe = jnp.dot(ef_ref[...], we_p, preferred_element_type=f32)
        part_n[...] = pn * LOG2E
        part_e[...] = pe * LOG2E
        partb_n[...] = pn * (0.01 * LOG2E)
        partb_e[...] = pe * (0.01 * LOG2E)
        mp_n[...] = jnp.max(pn, axis=0, keepdims=True)
        mp_e[...] = jnp.max(pe, axis=0, keepdims=True)

    pvn = pvn_ref[...]
    pve = pve_ref[...]
    we_n = lax.dot_general(we_ref[...], pvn[:, :ee], cd,
                           preferred_element_type=f32)
    we_b = lax.dot_general(we_ref[...], pve[:, :ee], cd,
                           preferred_element_type=f32)
    ef_i = ef_ref[pl.ds(i * BR, BR), :]
    base_n = lax.dot_general(we_n, ef_i, c0, preferred_element_type=f32)
    base_e = lax.dot_general(we_b, ef_i, c0, preferred_element_type=f32)
    m2_n = _leaky(base_n + mp_n[...]) * LOG2E
    m2_e = _leaky(base_e + mp_e[...]) * LOG2E
    d1_n = base_n * LOG2E - m2_n
    d2_n = base_n * (0.01 * LOG2E) - m2_n
    d1_e = base_e * LOG2E - m2_e
    d2_e = base_e * (0.01 * LOG2E) - m2_e

    nfw = nf_ref.shape[1]
    efw = ef_ref.shape[1]
    accl_n = jnp.zeros((nfw + 1, BR), f32)
    accl_e = jnp.zeros((efw + 1, BR), f32)
    ci_n = jnp.zeros((1, BR), jnp.int32)
    ci_e = jnp.zeros((1, BR), jnp.int32)

    for j in range(nj):
        sl = pl.ds(j * BC, BC)

        ai = n2e_ref[sl, :]
        t = jnp.maximum(part_n[sl, :] + d1_n, partb_n[sl, :] + d2_n)
        p = jnp.where(ai > 0, jnp.exp2(t), 0.0)
        ci_n += jnp.sum(ai, axis=0, keepdims=True)
        accl_n += lax.dot_general(nfb_ref[sl, :], p.astype(bf16),
                                  c00, preferred_element_type=f32)

        ai = e2e_ref[sl, :]
        t = jnp.maximum(part_e[sl, :] + d1_e, partb_e[sl, :] + d2_e)
        p = jnp.where(ai > 0, jnp.exp2(t), 0.0)
        ci_e += jnp.sum(ai, axis=0, keepdims=True)
        accl_e += lax.dot_general(efb_ref[sl, :], p.astype(bf16),
                                  c00, preferred_element_type=f32)

    c_n = ci_n.astype(f32)
    c_e = ci_e.astype(f32)
    means_n = accl_n[:nfw, :] / (accl_n[nfw:, :] * c_n)
    o_n = lax.dot_general(wn_ref[...], means_n, c00,
                          preferred_element_type=f32)
    out_n_ref[...] = jnp.transpose(_leaky(o_n))
    means_e = accl_e[:efw, :] / (accl_e[efw:, :] * c_e)
    o_e = lax.dot_general(we_ref[...], means_e, c00,
                          preferred_element_type=f32)
    out_e_ref[...] = jnp.transpose(_leaky(o_e))


def kernel(node_features, edge_features, edge_to_edge_adj_matrix,
           node_to_edge_adj_matrix, weight_node, weight_edge,
           parameter_vector_edge, parameter_vector_node):
    n, nf = node_features.shape
    e, ef = edge_features.shape
    ne = weight_node.shape[1]
    ee = weight_edge.shape[1]
    ni = e // BR
    nj = n // BC

    pvn = parameter_vector_node.reshape(1, -1)
    pve = parameter_vector_edge.reshape(1, -1)
    bf16 = jnp.bfloat16
    nfb = jnp.concatenate(
        [node_features.astype(bf16), jnp.ones((n, 1), bf16)], axis=1)
    efb = jnp.concatenate(
        [edge_features.astype(bf16), jnp.ones((e, 1), bf16)], axis=1)

    out_nodes, out_edges = pl.pallas_call(
        functools.partial(_attn_kernel, nj=nj, ee=ee),
        grid=(ni,),
        in_specs=[
            pl.BlockSpec((n, BR), lambda i: (0, i)),
            pl.BlockSpec((e, BR), lambda i: (0, i)),
            pl.BlockSpec((n, nf), lambda i: (0, 0)),
            pl.BlockSpec((e, ef), lambda i: (0, 0)),
            pl.BlockSpec((n, nf + 1), lambda i: (0, 0)),
            pl.BlockSpec((e, ef + 1), lambda i: (0, 0)),
            pl.BlockSpec((nf, ne), lambda i: (0, 0)),
            pl.BlockSpec((ef, ee), lambda i: (0, 0)),
            pl.BlockSpec((1, ee + ne), lambda i: (0, 0)),
            pl.BlockSpec((1, 2 * ee), lambda i: (0, 0)),
        ],
        out_specs=[
            pl.BlockSpec((BR, ne), lambda i: (i, 0)),
            pl.BlockSpec((BR, ee), lambda i: (i, 0)),
        ],
        out_shape=[
            jax.ShapeDtypeStruct((e, ne), jnp.float32),
            jax.ShapeDtypeStruct((e, ee), jnp.float32),
        ],
        scratch_shapes=[
            pltpu.VMEM((n, 1), jnp.float32),
            pltpu.VMEM((e, 1), jnp.float32),
            pltpu.VMEM((n, 1), jnp.float32),
            pltpu.VMEM((e, 1), jnp.float32),
            pltpu.VMEM((1, 1), jnp.float32),
            pltpu.VMEM((1, 1), jnp.float32),
        ],
    )(node_to_edge_adj_matrix, edge_to_edge_adj_matrix, node_features,
      edge_features, nfb, efb, weight_node, weight_edge, pvn, pve)

    return jnp.concatenate([out_nodes, out_edges], axis=1)


# speedup vs baseline: 1.2495x; 1.2495x over previous
import functools

import jax
import jax.numpy as jnp
from jax import lax
from jax.experimental import pallas as pl
from jax.experimental.pallas import tpu as pltpu

BR = 512
BC = 1024
LOG2E = 1.4426950408889634


def _leaky(x):
    return jnp.maximum(x, 0.01 * x)


def _attn_kernel(n2e_ref, e2e_ref, nf_ref, ef_ref, nfb_ref, efb_ref,
                 wn_ref, we_ref, pvn_ref, pve_ref, out_n_ref, out_e_ref,
                 part_n, part_e, mp_n, mp_e,
                 *, nj, ee):
    i = pl.program_id(0)
    cd = (((1,), (1,)), ((), ()))
    c0 = (((0,), (1,)), ((), ()))
    c00 = (((0,), (0,)), ((), ()))
    f32 = jnp.float32
    bf16 = jnp.bfloat16

    @pl.when(i == 0)
    def _precompute():
        wn_vec = lax.dot_general(wn_ref[...], pvn_ref[...][:, ee:], cd,
                                 preferred_element_type=f32)
        we_p = lax.dot_general(we_ref[...], pve_ref[...][:, ee:], cd,
                               preferred_element_type=f32)
        pn = jnp.dot(nf_ref[...], wn_vec, preferred_element_type=f32)
        pe = jnp.dot(ef_ref[...], we_p, preferred_element_type=f32)
        part_n[...] = pn * LOG2E
        part_e[...] = pe * LOG2E
        mp_n[...] = jnp.max(pn, axis=0, keepdims=True)
        mp_e[...] = jnp.max(pe, axis=0, keepdims=True)

    pvn = pvn_ref[...]
    pve = pve_ref[...]
    we_n = lax.dot_general(we_ref[...], pvn[:, :ee], cd,
                           preferred_element_type=f32)
    we_b = lax.dot_general(we_ref[...], pve[:, :ee], cd,
                           preferred_element_type=f32)
    ef_i = ef_ref[pl.ds(i * BR, BR), :]
    base_n = lax.dot_general(we_n, ef_i, c0, preferred_element_type=f32)
    base_e = lax.dot_general(we_b, ef_i, c0, preferred_element_type=f32)
    m2_n = _leaky(base_n + mp_n[...]) * LOG2E
    m2_e = _leaky(base_e + mp_e[...]) * LOG2E
    b2_n = base_n * LOG2E
    b2_e = base_e * LOG2E

    nfw = nf_ref.shape[1]
    efw = ef_ref.shape[1]
    accl_n = jnp.zeros((nfw + 1, BR), f32)
    accl_e = jnp.zeros((efw + 1, BR), f32)
    ci_n = jnp.zeros((1, BR), jnp.int32)
    ci_e = jnp.zeros((1, BR), jnp.int32)

    for j in range(nj):
        sl = pl.ds(j * BC, BC)

        ai = n2e_ref[sl, :]
        x2 = part_n[sl, :] + b2_n
        p = jnp.where(ai > 0, jnp.exp2(_leaky(x2) - m2_n), 0.0)
        ci_n += jnp.sum(ai, axis=0, keepdims=True)
        accl_n += lax.dot_general(nfb_ref[sl, :], p.astype(bf16),
                                  c00, preferred_element_type=f32)

        ai = e2e_ref[sl, :]
        x2 = part_e[sl, :] + b2_e
        p = jnp.where(ai > 0, jnp.exp2(_leaky(x2) - m2_e), 0.0)
        ci_e += jnp.sum(ai, axis=0, keepdims=True)
        accl_e += lax.dot_general(efb_ref[sl, :], p.astype(bf16),
                                  c00, preferred_element_type=f32)

    c_n = ci_n.astype(f32)
    c_e = ci_e.astype(f32)
    means_n = accl_n[:nfw, :] / (accl_n[nfw:, :] * c_n)
    o_n = lax.dot_general(wn_ref[...], means_n, c00,
                          preferred_element_type=f32)
    out_n_ref[...] = jnp.transpose(_leaky(o_n))
    means_e = accl_e[:efw, :] / (accl_e[efw:, :] * c_e)
    o_e = lax.dot_general(we_ref[...], means_e, c00,
                          preferred_element_type=f32)
    out_e_ref[...] = jnp.transpose(_leaky(o_e))


def kernel(node_features, edge_features, edge_to_edge_adj_matrix,
           node_to_edge_adj_matrix, weight_node, weight_edge,
           parameter_vector_edge, parameter_vector_node):
    n, nf = node_features.shape
    e, ef = edge_features.shape
    ne = weight_node.shape[1]
    ee = weight_edge.shape[1]
    ni = e // BR
    nj = n // BC

    pvn = parameter_vector_node.reshape(1, -1)
    pve = parameter_vector_edge.reshape(1, -1)
    bf16 = jnp.bfloat16
    nfb = jnp.concatenate(
        [node_features.astype(bf16), jnp.ones((n, 1), bf16)], axis=1)
    efb = jnp.concatenate(
        [edge_features.astype(bf16), jnp.ones((e, 1), bf16)], axis=1)

    out_nodes, out_edges = pl.pallas_call(
        functools.partial(_attn_kernel, nj=nj, ee=ee),
        grid=(ni,),
        in_specs=[
            pl.BlockSpec((n, BR), lambda i: (0, i)),
            pl.BlockSpec((e, BR), lambda i: (0, i)),
            pl.BlockSpec((n, nf), lambda i: (0, 0)),
            pl.BlockSpec((e, ef), lambda i: (0, 0)),
            pl.BlockSpec((n, nf + 1), lambda i: (0, 0)),
            pl.BlockSpec((e, ef + 1), lambda i: (0, 0)),
            pl.BlockSpec((nf, ne), lambda i: (0, 0)),
            pl.BlockSpec((ef, ee), lambda i: (0, 0)),
            pl.BlockSpec((1, ee + ne), lambda i: (0, 0)),
            pl.BlockSpec((1, 2 * ee), lambda i: (0, 0)),
        ],
        out_specs=[
            pl.BlockSpec((BR, ne), lambda i: (i, 0)),
            pl.BlockSpec((BR, ee), lambda i: (i, 0)),
        ],
        out_shape=[
            jax.ShapeDtypeStruct((e, ne), jnp.float32),
            jax.ShapeDtypeStruct((e, ee), jnp.float32),
        ],
        scratch_shapes=[
            pltpu.VMEM((n, 1), jnp.float32),
            pltpu.VMEM((e, 1), jnp.float32),
            pltpu.VMEM((1, 1), jnp.float32),
            pltpu.VMEM((1, 1), jnp.float32),
        ],
    )(node_to_edge_adj_matrix, edge_to_edge_adj_matrix, node_features,
      edge_features, nfb, efb, weight_node, weight_edge, pvn, pve)

    return jnp.concatenate([out_nodes, out_edges], axis=1)


# speedup vs baseline: 1.3700x; 1.0964x over previous
import functools

import jax
import jax.numpy as jnp
from jax import lax
from jax.experimental import pallas as pl
from jax.experimental.pallas import tpu as pltpu

BR = 512
BC = 1024
LOG2E = 1.4426950408889634


def _leaky(x):
    return jnp.maximum(x, 0.01 * x)


def _attn_kernel(n2e_ref, e2e_ref, nf_ref, ef_ref,
                 wn_ref, we_ref, pvn_ref, pve_ref, out_ref,
                 part_n, part_e, nfb_ref, efb_ref, mp_n, mp_e,
                 *, nj, ee):
    i = pl.program_id(0)
    cd = (((1,), (1,)), ((), ()))
    c0 = (((0,), (1,)), ((), ()))
    c00 = (((0,), (0,)), ((), ()))
    f32 = jnp.float32
    bf16 = jnp.bfloat16

    @pl.when(i == 0)
    def _precompute():
        wn_vec = lax.dot_general(wn_ref[...], pvn_ref[...][:, ee:], cd,
                                 preferred_element_type=f32)
        we_p = lax.dot_general(we_ref[...], pve_ref[...][:, ee:], cd,
                               preferred_element_type=f32)
        pn = jnp.dot(nf_ref[...], wn_vec, preferred_element_type=f32)
        pe = jnp.dot(ef_ref[...], we_p, preferred_element_type=f32)
        part_n[...] = pn * LOG2E
        part_e[...] = pe * LOG2E
        mp_n[...] = jnp.max(pn, axis=0, keepdims=True)
        mp_e[...] = jnp.max(pe, axis=0, keepdims=True)
        nw = nf_ref.shape[1]
        ew = ef_ref.shape[1]
        nfb_ref[:, :nw] = nf_ref[...].astype(bf16)
        nfb_ref[:, nw:] = jnp.ones((nfb_ref.shape[0], 1), bf16)
        efb_ref[:, :ew] = ef_ref[...].astype(bf16)
        efb_ref[:, ew:] = jnp.ones((efb_ref.shape[0], 1), bf16)

    pvn = pvn_ref[...]
    pve = pve_ref[...]
    we_n = lax.dot_general(we_ref[...], pvn[:, :ee], cd,
                           preferred_element_type=f32)
    we_b = lax.dot_general(we_ref[...], pve[:, :ee], cd,
                           preferred_element_type=f32)
    ef_i = ef_ref[pl.ds(i * BR, BR), :]
    base_n = lax.dot_general(we_n, ef_i, c0, preferred_element_type=f32)
    base_e = lax.dot_general(we_b, ef_i, c0, preferred_element_type=f32)
    m2_n = _leaky(base_n + mp_n[...]) * LOG2E
    m2_e = _leaky(base_e + mp_e[...]) * LOG2E
    b2_n = base_n * LOG2E
    b2_e = base_e * LOG2E

    nfw = nf_ref.shape[1]
    efw = ef_ref.shape[1]
    accl_n = jnp.zeros((nfw + 1, BR), f32)
    accl_e = jnp.zeros((efw + 1, BR), f32)
    ci_n = jnp.zeros((1, BR), jnp.int32)
    ci_e = jnp.zeros((1, BR), jnp.int32)

    for j in range(nj):
        sl = pl.ds(j * BC, BC)

        ai = n2e_ref[sl, :]
        x2 = part_n[sl, :] + b2_n
        p = jnp.where(ai > 0, jnp.exp2(_leaky(x2) - m2_n), 0.0)
        ci_n += jnp.sum(ai, axis=0, keepdims=True)
        accl_n += lax.dot_general(nfb_ref[sl, :], p.astype(bf16),
                                  c00, preferred_element_type=f32)

        ai = e2e_ref[sl, :]
        x2 = part_e[sl, :] + b2_e
        p = jnp.where(ai > 0, jnp.exp2(_leaky(x2) - m2_e), 0.0)
        ci_e += jnp.sum(ai, axis=0, keepdims=True)
        accl_e += lax.dot_general(efb_ref[sl, :], p.astype(bf16),
                                  c00, preferred_element_type=f32)

    c_n = ci_n.astype(f32)
    c_e = ci_e.astype(f32)
    means_n = accl_n[:nfw, :] / (accl_n[nfw:, :] * c_n)
    o_n = lax.dot_general(wn_ref[...], means_n, c00,
                          preferred_element_type=f32)
    newidth = o_n.shape[0]
    out_ref[:, :newidth] = jnp.transpose(_leaky(o_n))
    means_e = accl_e[:efw, :] / (accl_e[efw:, :] * c_e)
    o_e = lax.dot_general(we_ref[...], means_e, c00,
                          preferred_element_type=f32)
    out_ref[:, newidth:] = jnp.transpose(_leaky(o_e))


def kernel(node_features, edge_features, edge_to_edge_adj_matrix,
           node_to_edge_adj_matrix, weight_node, weight_edge,
           parameter_vector_edge, parameter_vector_node):
    n, nf = node_features.shape
    e, ef = edge_features.shape
    ne = weight_node.shape[1]
    ee = weight_edge.shape[1]
    ni = e // BR
    nj = n // BC

    pvn = parameter_vector_node.reshape(1, -1)
    pve = parameter_vector_edge.reshape(1, -1)

    return pl.pallas_call(
        functools.partial(_attn_kernel, nj=nj, ee=ee),
        grid=(ni,),
        in_specs=[
            pl.BlockSpec((n, BR), lambda i: (0, i)),
            pl.BlockSpec((e, BR), lambda i: (0, i)),
            pl.BlockSpec((n, nf), lambda i: (0, 0)),
            pl.BlockSpec((e, ef), lambda i: (0, 0)),
            pl.BlockSpec((nf, ne), lambda i: (0, 0)),
            pl.BlockSpec((ef, ee), lambda i: (0, 0)),
            pl.BlockSpec((1, ee + ne), lambda i: (0, 0)),
            pl.BlockSpec((1, 2 * ee), lambda i: (0, 0)),
        ],
        out_specs=pl.BlockSpec((BR, ne + ee), lambda i: (i, 0)),
        out_shape=jax.ShapeDtypeStruct((e, ne + ee), jnp.float32),
        scratch_shapes=[
            pltpu.VMEM((n, 1), jnp.float32),
            pltpu.VMEM((e, 1), jnp.float32),
            pltpu.VMEM((n, nf + 1), jnp.bfloat16),
            pltpu.VMEM((e, ef + 1), jnp.bfloat16),
            pltpu.VMEM((1, 1), jnp.float32),
            pltpu.VMEM((1, 1), jnp.float32),
        ],
    )(node_to_edge_adj_matrix, edge_to_edge_adj_matrix, node_features,
      edge_features, weight_node, weight_edge, pvn, pve)
